# baseline (speedup 1.0000x reference)
import functools

import jax
import jax.numpy as jnp
from jax import lax
from jax.experimental import pallas as pl
from jax.experimental.pallas import tpu as pltpu
from jax.experimental.pallas import tpu_sc as plsc

N = 10000
D = 128
E = 320000
NC = 2
NS = 16
NW = NC * NS
EPW = E // NW
CH = 128
NCG = E // CH
NCHUNK = NCG // NW
NXTRA = NCG - NCHUNK * NW
CA, CB = 25, 100
NBUF = 3
NIR = 6
DCH = 80
DNCH = EPW // DCH
RPT = 640
RLAST = N - 15 * RPT
BN = 1000
NPAD = 10240


def _sc_degree_body(dst_hbm, ones_hbm, zeros1_hbm, deg_out, idxd, ones_v,
                    dbounce, dacc):
    c = lax.axis_index("c")
    s = lax.axis_index("s")
    w = s * NC + c

    @pl.when(s == 0)
    def _():
        pltpu.sync_copy(zeros1_hbm, dbounce)
        pltpu.sync_copy(dbounce, dacc)

    pltpu.sync_copy(ones_hbm, ones_v)
    pltpu.sync_copy(dst_hbm.at[w], idxd)
    plsc.subcore_barrier()

    def body(j, carry):
        pltpu.sync_copy(ones_v, dacc.at[idxd.at[j]], add=True)
        return carry

    lax.fori_loop(0, DNCH, body, 0)
    plsc.subcore_barrier()

    @pl.when(s == 0)
    def _():
        pltpu.sync_copy(dacc, dbounce)
        pltpu.sync_copy(dbounce, deg_out.at[pl.ds(c * NPAD, N)])


@functools.cache
def _sc_degree():
    mesh = plsc.VectorSubcoreMesh(core_axis_name="c", subcore_axis_name="s")
    return pl.kernel(
        _sc_degree_body,
        out_type=jax.ShapeDtypeStruct((NC * NPAD,), jnp.float32),
        scratch_types=[
            pltpu.VMEM((DNCH, DCH), jnp.int32),
            pltpu.VMEM((DCH,), jnp.float32),
            pltpu.VMEM((N,), jnp.float32),
            pltpu.VMEM_SHARED((N,), jnp.float32),
        ],
        mesh=mesh,
    )


def _sc_prop_body(srcm_hbm, dstm_hbm, u_hbm, zeros_hbm,
                  out_hbm, sring, dring, rows, acc, *sems):
    gsem = sems[:NBUF]
    ssem = sems[NBUF:2 * NBUF]
    isem = sems[2 * NBUF:]
    c = lax.axis_index("c")
    s = lax.axis_index("s")
    w = s * NC + c
    r0 = s * RPT
    base = NCHUNK * w + jnp.minimum(w, NXTRA)

    @pl.when(s < 15)
    def _():
        pltpu.sync_copy(zeros_hbm, acc.at[pl.ds(r0, RPT)])

    @pl.when(s == 15)
    def _():
        pltpu.sync_copy(zeros_hbm.at[pl.ds(0, RLAST)],
                        acc.at[pl.ds(r0, RLAST)])

    plsc.subcore_barrier()

    def i_start(j, k):
        g = base + j
        pltpu.async_copy(srcm_hbm.at[g // CB, g % CB],
                         sring.at[pl.ds(k * CH, CH)], isem[k])
        pltpu.async_copy(dstm_hbm.at[g // CB, g % CB], dring.at[k], isem[k])

    def i_wait(j, k):
        g = base + j
        pltpu.make_async_copy(srcm_hbm.at[g // CB, g % CB],
                              sring.at[pl.ds(k * CH, CH)], isem[k]).wait()
        pltpu.make_async_copy(dstm_hbm.at[g // CB, g % CB], dring.at[k],
                              isem[k]).wait()

    def g_start(j, b, k):
        del j
        pltpu.async_copy(u_hbm.at[sring.at[pl.ds(k * CH, CH)]],
                         rows.at[b], gsem[b])

    def g_wait(j, b, k):
        del j
        pltpu.make_async_copy(u_hbm.at[sring.at[pl.ds(k * CH, CH)]],
                              rows.at[b], gsem[b]).wait()

    def s_start(j, b, k):
        del j
        pltpu.async_copy(rows.at[b], acc.at[dring.at[k]], ssem[b], add=True)

    def s_wait(j, b, k):
        del j
        pltpu.make_async_copy(rows.at[b], acc.at[dring.at[k]],
                              ssem[b]).wait()

    def round(j, jm, *, first=False, fetch=True, gnext=True):
        b = jm % NBUF
        g_wait(j, b, jm)
        s_start(j, b, jm)
        if fetch:
            i_start(j + 4, (jm + 4) % NIR)
        if not first:
            s_wait(j - 1, (b + 2) % NBUF, (jm + 5) % NIR)
        if gnext:
            i_wait(j + 2, (jm + 2) % NIR)
            g_start(j + 2, (jm + 2) % NBUF, (jm + 2) % NIR)

    for k in range(4):
        i_start(k, k)
    i_wait(0, 0)
    g_start(0, 0, 0)
    i_wait(1, 1)
    g_start(1, 1, 1)
    for j in range(NIR):
        round(j, j, first=(j == 0))

    def group_body(i, carry):
        for k in range(NIR):
            round(i * NIR + k, k)
        return carry

    ngroup = NCHUNK // NIR
    lax.fori_loop(1, ngroup - 1, group_body, 0)
    for j in range((ngroup - 1) * NIR, NCHUNK):
        round(j, j % NIR, fetch=(j + 4 < NCHUNK), gnext=(j + 2 < NCHUNK))
    s_wait(NCHUNK - 1, (NCHUNK - 1) % NBUF, (NCHUNK - 1) % NIR)

    gx = base + NCHUNK

    @pl.when(w < NXTRA)
    def _():
        pltpu.sync_copy(srcm_hbm.at[gx // CB, gx % CB],
                        sring.at[pl.ds(0, CH)])
        pltpu.sync_copy(dstm_hbm.at[gx // CB, gx % CB], dring.at[0])
        pltpu.async_copy(u_hbm.at[sring.at[pl.ds(0, CH)]], rows.at[0],
                        gsem[0]).wait()
        pltpu.sync_copy(rows.at[0], acc.at[dring.at[0]], add=True)

    plsc.subcore_barrier()

    @pl.when(s < 15)
    def _():
        pltpu.sync_copy(acc.at[pl.ds(r0, RPT)],
                        out_hbm.at[c, pl.ds(r0, RPT)])

    @pl.when(s == 15)
    def _():
        pltpu.sync_copy(acc.at[pl.ds(r0, RLAST)],
                        out_hbm.at[c, pl.ds(r0, RLAST)])


@functools.cache
def _sc_propagate():
    mesh = plsc.VectorSubcoreMesh(core_axis_name="c", subcore_axis_name="s")
    return pl.kernel(
        _sc_prop_body,
        out_type=jax.ShapeDtypeStruct((NC, N, D), jnp.float32),
        scratch_types=[
            pltpu.VMEM((NIR * CH,), jnp.int32),
            pltpu.VMEM((NIR, CH), jnp.int32),
            pltpu.VMEM((NBUF, CH, D), jnp.float32),
            pltpu.VMEM_SHARED((N, D), jnp.float32),
        ] + [pltpu.SemaphoreType.DMA] * (2 * NBUF + NIR),
        mesh=mesh,
    )


def _norm_body(degp, feat, norm, norm2, u0):
    d = degp[0, :, :] + degp[1, :, :]
    d = jnp.maximum(d, 1.0)
    nr = lax.rsqrt(d)
    norm[...] = nr
    norm2[...] = 1.0 / d
    u0[...] = feat[...] * nr


def _norm_kernel(degp, feat):
    grid = (N // BN,)
    return pl.pallas_call(
        _norm_body,
        grid=grid,
        in_specs=[
            pl.BlockSpec((NC, BN, 1), lambda i: (0, i, 0)),
            pl.BlockSpec((BN, D), lambda i: (i, 0)),
        ],
        out_specs=(
            pl.BlockSpec((BN, 1), lambda i: (i, 0)),
            pl.BlockSpec((BN, 1), lambda i: (i, 0)),
            pl.BlockSpec((BN, D), lambda i: (i, 0)),
        ),
        out_shape=(
            jax.ShapeDtypeStruct((N, 1), jnp.float32),
            jax.ShapeDtypeStruct((N, 1), jnp.float32),
            jax.ShapeDtypeStruct((N, D), jnp.float32),
        ),
    )(degp, feat)


def _scale_body(part, norm2, s_out, u_out):
    s = part[0, :, :] + part[1, :, :]
    s_out[...] = s
    u_out[...] = s * norm2[...]


def _scale(part, norm2):
    grid = (N // BN,)
    return pl.pallas_call(
        _scale_body,
        grid=grid,
        in_specs=[
            pl.BlockSpec((NC, BN, D), lambda i: (0, i, 0)),
            pl.BlockSpec((BN, 1), lambda i: (i, 0)),
        ],
        out_specs=(
            pl.BlockSpec((BN, D), lambda i: (i, 0)),
            pl.BlockSpec((BN, D), lambda i: (i, 0)),
        ),
        out_shape=(
            jax.ShapeDtypeStruct((N, D), jnp.float32),
            jax.ShapeDtypeStruct((N, D), jnp.float32),
        ),
    )(part, norm2)


def _fold_body(w0, w1, w2, wfc, b0, b1, b2, bfc, mt, btot):
    ws = [w0[...], w1[...], w2[...]]
    bs = [b0[...], b1[...], b2[...]]
    wf = wfc[...]
    acc = bfc[...]
    for j in range(3):
        wfj = wf[:, j * D:(j + 1) * D]
        mt[j * D:(j + 1) * D, :] = lax.dot_general(
            ws[j], wfj, (((0,), (1,)), ((), ())),
            preferred_element_type=jnp.float32)
        acc = acc + lax.dot_general(
            bs[j], wfj, (((1,), (1,)), ((), ())),
            preferred_element_type=jnp.float32)
    btot[...] = acc


def _fold(w0, w1, w2, wfc, b0, b1, b2, bfc):
    return pl.pallas_call(
        _fold_body,
        out_shape=(
            jax.ShapeDtypeStruct((3 * D, D), jnp.float32),
            jax.ShapeDtypeStruct((1, D), jnp.float32),
        ),
    )(w0, w1, w2, wfc, b0, b1, b2, bfc)


def _final_body(s0, s1, p2, norm, mt, btot, out):
    s2 = p2[0, :, :] + p2[1, :, :]
    m = mt[...]
    acc = jnp.dot(s0[...], m[0:D, :], preferred_element_type=jnp.float32)
    acc += jnp.dot(s1[...], m[D:2 * D, :], preferred_element_type=jnp.float32)
    acc += jnp.dot(s2, m[2 * D:3 * D, :], preferred_element_type=jnp.float32)
    out[...] = norm[...] * acc + btot[...]


def _final(s0, s1, p2, norm, mt, btot):
    grid = (N // BN,)
    return pl.pallas_call(
        _final_body,
        grid=grid,
        in_specs=[
            pl.BlockSpec((BN, D), lambda i: (i, 0)),
            pl.BlockSpec((BN, D), lambda i: (i, 0)),
            pl.BlockSpec((NC, BN, D), lambda i: (0, i, 0)),
            pl.BlockSpec((BN, 1), lambda i: (i, 0)),
            pl.BlockSpec((3 * D, D), lambda i: (0, 0)),
            pl.BlockSpec((1, D), lambda i: (0, 0)),
        ],
        out_specs=pl.BlockSpec((BN, D), lambda i: (i, 0)),
        out_shape=jax.ShapeDtypeStruct((N, D), jnp.float32),
    )(s0, s1, p2, norm, mt, btot)


def kernel(feat, edge_index, W0, b0, W1, b1, W2, b2, Wfc, bfc):
    srcm = edge_index[0].reshape(CA, CB, CH)
    dstm = edge_index[1].reshape(CA, CB, CH)
    dst3d = edge_index[1].reshape(NW, DNCH, DCH)
    zeros2 = jnp.zeros((RPT, D), jnp.float32)
    ones1 = jnp.ones((DCH,), jnp.float32)
    zeros1 = jnp.zeros((N,), jnp.float32)

    dpad = _sc_degree()(dst3d, ones1, zeros1)
    degp = jnp.stack([dpad[0:N], dpad[NPAD:NPAD + N]])
    mt, btot = _fold(W0, W1, W2, Wfc, b0.reshape(1, D), b1.reshape(1, D),
                     b2.reshape(1, D), bfc.reshape(1, D))
    norm, norm2, u = _norm_kernel(degp.reshape(NC, N, 1), feat)

    prop = _sc_propagate()
    p = prop(srcm, dstm, u, zeros2)
    s0, u = _scale(p, norm2)
    p = prop(srcm, dstm, u, zeros2)
    s1, u = _scale(p, norm2)
    p2 = prop(srcm, dstm, u, zeros2)

    return _final(s0, s1, p2, norm, mt, btot)


# speedup vs baseline: 1.0102x; 1.0102x over previous
import functools

import jax
import jax.numpy as jnp
from jax import lax
from jax.experimental import pallas as pl
from jax.experimental.pallas import tpu as pltpu
from jax.experimental.pallas import tpu_sc as plsc

N = 10000
D = 128
E = 320000
NC = 2
NS = 16
NW = NC * NS
EPW = E // NW
CH = 128
NCG = E // CH
NCHUNK = NCG // NW
NXTRA = NCG - NCHUNK * NW
CA, CB = 25, 100
NBUF = 3
NIR = 6
DCH = 80
DNCH = EPW // DCH
NRT = 10
RPT = N // NRT
BN = 1000
NPAD = 10240


def _sc_degree_body(dst_hbm, ones_hbm, zeros1_hbm, deg_out, idxd, ones_v,
                    dbounce, dacc):
    c = lax.axis_index("c")
    s = lax.axis_index("s")
    w = s * NC + c

    @pl.when(s == 0)
    def _():
        pltpu.sync_copy(zeros1_hbm, dbounce)
        pltpu.sync_copy(dbounce, dacc)

    pltpu.sync_copy(ones_hbm, ones_v)
    pltpu.sync_copy(dst_hbm.at[w], idxd)
    plsc.subcore_barrier()

    def body(j, carry):
        pltpu.sync_copy(ones_v, dacc.at[idxd.at[j]], add=True)
        return carry

    lax.fori_loop(0, DNCH, body, 0)
    plsc.subcore_barrier()

    @pl.when(s == 0)
    def _():
        pltpu.sync_copy(dacc, dbounce)
        pltpu.sync_copy(dbounce, deg_out.at[pl.ds(c * NPAD, N)])


@functools.cache
def _sc_degree():
    mesh = plsc.VectorSubcoreMesh(core_axis_name="c", subcore_axis_name="s")
    return pl.kernel(
        _sc_degree_body,
        out_type=jax.ShapeDtypeStruct((NC * NPAD,), jnp.float32),
        scratch_types=[
            pltpu.VMEM((DNCH, DCH), jnp.int32),
            pltpu.VMEM((DCH,), jnp.float32),
            pltpu.VMEM((N,), jnp.float32),
            pltpu.VMEM_SHARED((N,), jnp.float32),
        ],
        mesh=mesh,
    )


def _sc_prop_body(srcm_hbm, dstm_hbm, u_hbm, zeros_hbm,
                  out_hbm, sring, dring, rows, acc, *sems):
    gsem = sems[:NBUF]
    ssem = sems[NBUF:2 * NBUF]
    isem = sems[2 * NBUF:]
    c = lax.axis_index("c")
    s = lax.axis_index("s")
    w = s * NC + c
    r0 = s * RPT
    base = NCHUNK * w + jnp.minimum(w, NXTRA)

    @pl.when(s < NRT)
    def _():
        pltpu.sync_copy(zeros_hbm, acc.at[pl.ds(r0, RPT)])

    plsc.subcore_barrier()

    def i_start(j, k):
        g = base + j
        pltpu.async_copy(srcm_hbm.at[g // CB, g % CB],
                         sring.at[pl.ds(k * CH, CH)], isem[k])
        pltpu.async_copy(dstm_hbm.at[g // CB, g % CB], dring.at[k], isem[k])

    def i_wait(j, k):
        g = base + j
        pltpu.make_async_copy(srcm_hbm.at[g // CB, g % CB],
                              sring.at[pl.ds(k * CH, CH)], isem[k]).wait()
        pltpu.make_async_copy(dstm_hbm.at[g // CB, g % CB], dring.at[k],
                              isem[k]).wait()

    def g_start(j, b, k):
        del j
        pltpu.async_copy(u_hbm.at[sring.at[pl.ds(k * CH, CH)]],
                         rows.at[b], gsem[b])

    def g_wait(j, b, k):
        del j
        pltpu.make_async_copy(u_hbm.at[sring.at[pl.ds(k * CH, CH)]],
                              rows.at[b], gsem[b]).wait()

    def s_start(j, b, k):
        del j
        pltpu.async_copy(rows.at[b], acc.at[dring.at[k]], ssem[b], add=True)

    def s_wait(j, b, k):
        del j
        pltpu.make_async_copy(rows.at[b], acc.at[dring.at[k]],
                              ssem[b]).wait()

    def round(j, jm, *, first=False, fetch=True, gnext=True):
        b = jm % NBUF
        g_wait(j, b, jm)
        s_start(j, b, jm)
        if fetch:
            i_start(j + 4, (jm + 4) % NIR)
        if not first:
            s_wait(j - 1, (b + 2) % NBUF, (jm + 5) % NIR)
        if gnext:
            i_wait(j + 2, (jm + 2) % NIR)
            g_start(j + 2, (jm + 2) % NBUF, (jm + 2) % NIR)

    for k in range(4):
        i_start(k, k)
    i_wait(0, 0)
    g_start(0, 0, 0)
    i_wait(1, 1)
    g_start(1, 1, 1)
    for j in range(NIR):
        round(j, j, first=(j == 0))

    def group_body(i, carry):
        for k in range(NIR):
            round(i * NIR + k, k)
        return carry

    ngroup = NCHUNK // NIR
    lax.fori_loop(1, ngroup - 1, group_body, 0)
    for j in range((ngroup - 1) * NIR, NCHUNK):
        round(j, j % NIR, fetch=(j + 4 < NCHUNK), gnext=(j + 2 < NCHUNK))
    s_wait(NCHUNK - 1, (NCHUNK - 1) % NBUF, (NCHUNK - 1) % NIR)

    gx = base + NCHUNK

    @pl.when(w < NXTRA)
    def _():
        pltpu.sync_copy(srcm_hbm.at[gx // CB, gx % CB],
                        sring.at[pl.ds(0, CH)])
        pltpu.sync_copy(dstm_hbm.at[gx // CB, gx % CB], dring.at[0])
        pltpu.async_copy(u_hbm.at[sring.at[pl.ds(0, CH)]], rows.at[0],
                        gsem[0]).wait()
        pltpu.sync_copy(rows.at[0], acc.at[dring.at[0]], add=True)

    plsc.subcore_barrier()

    @pl.when(s < NRT)
    def _():
        pltpu.sync_copy(acc.at[pl.ds(r0, RPT)],
                        out_hbm.at[c, pl.ds(r0, RPT)])


@functools.cache
def _sc_propagate():
    mesh = plsc.VectorSubcoreMesh(core_axis_name="c", subcore_axis_name="s")
    return pl.kernel(
        _sc_prop_body,
        out_type=jax.ShapeDtypeStruct((NC, N, D), jnp.float32),
        scratch_types=[
            pltpu.VMEM((NIR * CH,), jnp.int32),
            pltpu.VMEM((NIR, CH), jnp.int32),
            pltpu.VMEM((NBUF, CH, D), jnp.float32),
            pltpu.VMEM_SHARED((N, D), jnp.float32),
        ] + [pltpu.SemaphoreType.DMA] * (2 * NBUF + NIR),
        mesh=mesh,
    )


def _norm_body(degp, feat, norm, norm2, u0):
    d = degp[0, :, :] + degp[1, :, :]
    d = jnp.maximum(d, 1.0)
    nr = lax.rsqrt(d)
    norm[...] = nr
    norm2[...] = 1.0 / d
    u0[...] = feat[...] * nr


def _norm_kernel(degp, feat):
    grid = (N // BN,)
    return pl.pallas_call(
        _norm_body,
        grid=grid,
        in_specs=[
            pl.BlockSpec((NC, BN, 1), lambda i: (0, i, 0)),
            pl.BlockSpec((BN, D), lambda i: (i, 0)),
        ],
        out_specs=(
            pl.BlockSpec((BN, 1), lambda i: (i, 0)),
            pl.BlockSpec((BN, 1), lambda i: (i, 0)),
            pl.BlockSpec((BN, D), lambda i: (i, 0)),
        ),
        out_shape=(
            jax.ShapeDtypeStruct((N, 1), jnp.float32),
            jax.ShapeDtypeStruct((N, 1), jnp.float32),
            jax.ShapeDtypeStruct((N, D), jnp.float32),
        ),
    )(degp, feat)


def _scale_body(part, norm2, s_out, u_out):
    s = part[0, :, :] + part[1, :, :]
    s_out[...] = s
    u_out[...] = s * norm2[...]


def _scale(part, norm2):
    grid = (N // BN,)
    return pl.pallas_call(
        _scale_body,
        grid=grid,
        in_specs=[
            pl.BlockSpec((NC, BN, D), lambda i: (0, i, 0)),
            pl.BlockSpec((BN, 1), lambda i: (i, 0)),
        ],
        out_specs=(
            pl.BlockSpec((BN, D), lambda i: (i, 0)),
            pl.BlockSpec((BN, D), lambda i: (i, 0)),
        ),
        out_shape=(
            jax.ShapeDtypeStruct((N, D), jnp.float32),
            jax.ShapeDtypeStruct((N, D), jnp.float32),
        ),
    )(part, norm2)


def _fold_body(w0, w1, w2, wfc, b0, b1, b2, bfc, mt, btot):
    ws = [w0[...], w1[...], w2[...]]
    bs = [b0[...], b1[...], b2[...]]
    wf = wfc[...]
    acc = bfc[...]
    for j in range(3):
        wfj = wf[:, j * D:(j + 1) * D]
        mt[j * D:(j + 1) * D, :] = lax.dot_general(
            ws[j], wfj, (((0,), (1,)), ((), ())),
            preferred_element_type=jnp.float32)
        acc = acc + lax.dot_general(
            bs[j], wfj, (((1,), (1,)), ((), ())),
            preferred_element_type=jnp.float32)
    btot[...] = acc


def _fold(w0, w1, w2, wfc, b0, b1, b2, bfc):
    return pl.pallas_call(
        _fold_body,
        out_shape=(
            jax.ShapeDtypeStruct((3 * D, D), jnp.float32),
            jax.ShapeDtypeStruct((1, D), jnp.float32),
        ),
    )(w0, w1, w2, wfc, b0, b1, b2, bfc)


def _final_body(s0, s1, p2, norm, mt, btot, out):
    s2 = p2[0, :, :] + p2[1, :, :]
    m = mt[...]
    acc = jnp.dot(s0[...], m[0:D, :], preferred_element_type=jnp.float32)
    acc += jnp.dot(s1[...], m[D:2 * D, :], preferred_element_type=jnp.float32)
    acc += jnp.dot(s2, m[2 * D:3 * D, :], preferred_element_type=jnp.float32)
    out[...] = norm[...] * acc + btot[...]


def _final(s0, s1, p2, norm, mt, btot):
    grid = (N // BN,)
    return pl.pallas_call(
        _final_body,
        grid=grid,
        in_specs=[
            pl.BlockSpec((BN, D), lambda i: (i, 0)),
            pl.BlockSpec((BN, D), lambda i: (i, 0)),
            pl.BlockSpec((NC, BN, D), lambda i: (0, i, 0)),
            pl.BlockSpec((BN, 1), lambda i: (i, 0)),
            pl.BlockSpec((3 * D, D), lambda i: (0, 0)),
            pl.BlockSpec((1, D), lambda i: (0, 0)),
        ],
        out_specs=pl.BlockSpec((BN, D), lambda i: (i, 0)),
        out_shape=jax.ShapeDtypeStruct((N, D), jnp.float32),
    )(s0, s1, p2, norm, mt, btot)


def kernel(feat, edge_index, W0, b0, W1, b1, W2, b2, Wfc, bfc):
    srcm = edge_index[0].reshape(CA, CB, CH)
    dstm = edge_index[1].reshape(CA, CB, CH)
    dst3d = edge_index[1].reshape(NW, DNCH, DCH)
    zeros2 = jnp.zeros((RPT, D), jnp.float32)
    ones1 = jnp.ones((DCH,), jnp.float32)
    zeros1 = jnp.zeros((N,), jnp.float32)

    dpad = _sc_degree()(dst3d, ones1, zeros1)
    degp = jnp.stack([dpad[0:N], dpad[NPAD:NPAD + N]])
    mt, btot = _fold(W0, W1, W2, Wfc, b0.reshape(1, D), b1.reshape(1, D),
                     b2.reshape(1, D), bfc.reshape(1, D))
    norm, norm2, u = _norm_kernel(degp.reshape(NC, N, 1), feat)

    prop = _sc_propagate()
    p = prop(srcm, dstm, u, zeros2)
    s0, u = _scale(p, norm2)
    p = prop(srcm, dstm, u, zeros2)
    s1, u = _scale(p, norm2)
    p2 = prop(srcm, dstm, u, zeros2)

    return _final(s0, s1, p2, norm, mt, btot)
